# baseline (speedup 1.0000x reference)
import functools

import jax
import jax.numpy as jnp
from jax import lax
from jax.experimental import pallas as pl
from jax.experimental.pallas import tpu as pltpu
from jax.experimental.pallas import tpu_sc as plsc

NC = 2
NS = 16
NW = NC * NS
NBUF = 3
LA = 2
LANES = 16


@functools.lru_cache(maxsize=None)
def _dual_gather(B: int, L: int, D: int):
    bpw = B // NW
    dg = D // 8
    nbb = B // 128
    n_iters = -(-L // NBUF) * NBUF
    mesh = plsc.VectorSubcoreMesh(core_axis_name="c", subcore_axis_name="s")

    o5 = jax.ShapeDtypeStruct((L, dg, nbb, 8, 128), jnp.float32)

    @functools.partial(
        pl.kernel,
        out_type=(o5, o5),
        mesh=mesh,
        scratch_types=[
            pltpu.VMEM((bpw * L,), jnp.int32),
            pltpu.VMEM((L, bpw), jnp.int32),
            pltpu.VMEM((NBUF, bpw, D), jnp.float32),
            pltpu.VMEM((NBUF, bpw, D), jnp.float32),
            pltpu.VMEM((NBUF, 1, dg, 1, 8, 128), jnp.float32),
            pltpu.VMEM((NBUF, 1, dg, 1, 8, 128), jnp.float32),
            pltpu.SemaphoreType.DMA((NBUF,)),
            pltpu.SemaphoreType.DMA((NBUF,)),
            pltpu.SemaphoreType.DMA((NBUF,)),
            pltpu.SemaphoreType.DMA((NBUF,)),
        ],
        compiler_params=pltpu.CompilerParams(use_tc_tiling_on_sc=False,
                                             needs_layout_passes=False),
    )
    def k(idx_hbm, amp_hbm, ph_hbm, amp_out, ph_out,
          idx_f, idx_t, abuf, pbuf, atr, ptr, ga_sem, gp_sem, oa_sem, op_sem):
        wid = lax.axis_index("s") * NC + lax.axis_index("c")

        pltpu.sync_copy(idx_hbm.at[pl.ds(wid * bpw * L, bpw * L)], idx_f)

        bvecs = [jnp.arange(LANES, dtype=jnp.int32) + LANES * kk
                 for kk in range(bpw // LANES)]

        def tr_idx(l, carry):
            for kk in range(bpw // LANES):
                v = plsc.load_gather(idx_f, [bvecs[kk] * L + l])
                idx_t[l, pl.ds(kk * LANES, LANES)] = v
            return carry

        lax.fori_loop(0, L, tr_idx, 0)

        def gather_descs(l, s):
            return (
                pltpu.make_async_copy(amp_hbm.at[idx_t.at[l]], abuf.at[s],
                                      ga_sem.at[s]),
                pltpu.make_async_copy(ph_hbm.at[idx_t.at[l]], pbuf.at[s],
                                      gp_sem.at[s]),
            )

        def out_descs(l, s):
            dst = (pl.ds(l, 1), pl.ds(0, dg), pl.ds(wid, 1),
                   pl.ds(0, 8), pl.ds(0, 128))
            return (
                pltpu.make_async_copy(atr.at[s], amp_out.at[dst], oa_sem.at[s]),
                pltpu.make_async_copy(ptr.at[s], ph_out.at[dst], op_sem.at[s]),
            )

        def transpose2(src_a, dst_a, src_p, dst_p):
            @plsc.parallel_loop(0, dg)
            def _per_g(g):
                base = jnp.broadcast_to(g * 8, (LANES,)).astype(jnp.int32)
                for dd in range(8):
                    col = base + dd
                    va = [plsc.load_gather(src_a, [bvecs[kk], col])
                          for kk in range(bpw // LANES)]
                    vp = [plsc.load_gather(src_p, [bvecs[kk], col])
                          for kk in range(bpw // LANES)]
                    for kk in range(bpw // LANES):
                        dst_a[0, g, 0, dd, pl.ds(kk * LANES, LANES)] = va[kk]
                        dst_p[0, g, 0, dd, pl.ds(kk * LANES, LANES)] = vp[kk]

        for l in range(LA):
            for d in gather_descs(l, l % NBUF):
                d.start()

        def group(g, carry):
            for s in range(NBUF):
                l = g * NBUF + s

                @pl.when(l < L)
                def _body():
                    da, dp = gather_descs(l, s)
                    da.wait()
                    dp.wait()

                    @pl.when(l >= NBUF)
                    def _drain():
                        poa, pop = out_descs(l - NBUF, s)
                        poa.wait()
                        pop.wait()

                    transpose2(abuf.at[s], atr.at[s], pbuf.at[s], ptr.at[s])
                    oa, op = out_descs(l, s)
                    oa.start()
                    op.start()

                    @pl.when(l + LA < L)
                    def _prefetch():
                        for d in gather_descs(l + LA, (s + LA) % NBUF):
                            d.start()
            return carry

        lax.fori_loop(0, n_iters // NBUF, group, 0)

        for l in range(L - NBUF, L):
            oa, op = out_descs(l, l % NBUF)
            oa.wait()
            op.wait()

    return k


def kernel(indices, amplitude_table, phase_table):
    B, L = indices.shape
    _, D = amplitude_table.shape
    flat = indices.reshape(B * L)
    o5a, o5p = _dual_gather(B, L, D)(flat, amplitude_table, phase_table)

    def finish(o):
        t = o.transpose(2, 4, 0, 1, 3)
        return t.reshape(B, L, D)

    return finish(o5a), finish(o5p)


# speedup vs baseline: 1.0284x; 1.0284x over previous
import functools

import jax
import jax.numpy as jnp
from jax import lax
from jax.experimental import pallas as pl
from jax.experimental.pallas import tpu as pltpu
from jax.experimental.pallas import tpu_sc as plsc

NC = 2
NS = 16
NW = NC * NS
NBUF = 3
LA = 2
LANES = 16


@functools.lru_cache(maxsize=None)
def _dual_gather(B: int, L: int, D: int):
    bpw = B // NW
    dg = D // 8
    nbb = B // 128
    n_iters = -(-L // NBUF) * NBUF
    mesh = plsc.VectorSubcoreMesh(core_axis_name="c", subcore_axis_name="s")

    o5 = jax.ShapeDtypeStruct((L, dg, nbb, 8, 128), jnp.float32)

    @functools.partial(
        pl.kernel,
        out_type=(o5, o5),
        mesh=mesh,
        scratch_types=[
            pltpu.VMEM((bpw * L,), jnp.int32),
            pltpu.VMEM((L, bpw), jnp.int32),
            pltpu.VMEM((NBUF, bpw, D), jnp.float32),
            pltpu.VMEM((NBUF, bpw, D), jnp.float32),
            pltpu.VMEM((NBUF, 1, dg, 1, 8, 128), jnp.float32),
            pltpu.VMEM((NBUF, 1, dg, 1, 8, 128), jnp.float32),
            pltpu.SemaphoreType.DMA((NBUF,)),
            pltpu.SemaphoreType.DMA((NBUF,)),
            pltpu.SemaphoreType.DMA((NBUF,)),
            pltpu.SemaphoreType.DMA((NBUF,)),
        ],
        compiler_params=pltpu.CompilerParams(use_tc_tiling_on_sc=False,
                                             needs_layout_passes=False),
    )
    def k(idx_hbm, amp_hbm, ph_hbm, amp_out, ph_out,
          idx_f, idx_t, abuf, pbuf, atr, ptr, ga_sem, gp_sem, oa_sem, op_sem):
        wid = lax.axis_index("s") * NC + lax.axis_index("c")

        pltpu.sync_copy(idx_hbm.at[pl.ds(wid * bpw * L, bpw * L)], idx_f)

        bvecs = [jnp.arange(LANES, dtype=jnp.int32) + LANES * kk
                 for kk in range(bpw // LANES)]

        def tr_idx(l, carry):
            for kk in range(bpw // LANES):
                v = plsc.load_gather(idx_f, [bvecs[kk] * L + l])
                idx_t[l, pl.ds(kk * LANES, LANES)] = v
            return carry

        lax.fori_loop(0, L, tr_idx, 0)

        def gather_descs(l, s):
            return (
                pltpu.make_async_copy(amp_hbm.at[idx_t.at[l]], abuf.at[s],
                                      ga_sem.at[s]),
                pltpu.make_async_copy(ph_hbm.at[idx_t.at[l]], pbuf.at[s],
                                      gp_sem.at[s]),
            )

        def out_descs(l, s):
            dst = (pl.ds(l, 1), pl.ds(0, dg), pl.ds(wid, 1),
                   pl.ds(0, 8), pl.ds(0, 128))
            return (
                pltpu.make_async_copy(atr.at[s], amp_out.at[dst], oa_sem.at[s]),
                pltpu.make_async_copy(ptr.at[s], ph_out.at[dst], op_sem.at[s]),
            )

        def transpose2(src_a, dst_a, src_p, dst_p):
            @plsc.parallel_loop(0, dg)
            def _per_g(g):
                base = jnp.broadcast_to(g * 8, (LANES,)).astype(jnp.int32)
                nk = bpw // LANES
                for src, dst in ((src_a, dst_a), (src_p, dst_p)):
                    for half in range(2):
                        vs = [[plsc.load_gather(src, [bvecs[kk],
                                                      base + (half * 4 + i)])
                               for kk in range(nk)] for i in range(4)]
                        for i in range(4):
                            for kk in range(nk):
                                dst[0, g, 0, half * 4 + i,
                                    pl.ds(kk * LANES, LANES)] = vs[i][kk]

        for l in range(LA):
            for d in gather_descs(l, l % NBUF):
                d.start()

        def group(g, carry):
            for s in range(NBUF):
                l = g * NBUF + s

                @pl.when(l < L)
                def _body():
                    da, dp = gather_descs(l, s)
                    da.wait()
                    dp.wait()

                    @pl.when(l >= NBUF)
                    def _drain():
                        poa, pop = out_descs(l - NBUF, s)
                        poa.wait()
                        pop.wait()

                    transpose2(abuf.at[s], atr.at[s], pbuf.at[s], ptr.at[s])
                    oa, op = out_descs(l, s)
                    oa.start()
                    op.start()

                    @pl.when(l + LA < L)
                    def _prefetch():
                        for d in gather_descs(l + LA, (s + LA) % NBUF):
                            d.start()
            return carry

        lax.fori_loop(0, n_iters // NBUF, group, 0)

        for l in range(L - NBUF, L):
            oa, op = out_descs(l, l % NBUF)
            oa.wait()
            op.wait()

    return k


def kernel(indices, amplitude_table, phase_table):
    B, L = indices.shape
    _, D = amplitude_table.shape
    flat = indices.reshape(B * L)
    o5a, o5p = _dual_gather(B, L, D)(flat, amplitude_table, phase_table)

    def finish(o):
        t = o.transpose(2, 4, 0, 1, 3)
        return t.reshape(B, L, D)

    return finish(o5a), finish(o5p)


# speedup vs baseline: 1.4488x; 1.4088x over previous
import functools

import jax
import jax.numpy as jnp
from jax import lax
from jax.experimental import pallas as pl
from jax.experimental.pallas import tpu as pltpu
from jax.experimental.pallas import tpu_sc as plsc

NC = 2
NS = 16
NW = NC * NS
CB = 2
NBUF = 4
LA = 2


@functools.lru_cache(maxsize=None)
def _dual_gather(B: int, L: int, D: int):
    bpw = B // NW
    n_chunks = bpw // CB
    n_groups = n_chunks // NBUF
    mesh = plsc.VectorSubcoreMesh(core_axis_name="c", subcore_axis_name="s")

    @functools.partial(
        pl.kernel,
        out_type=(
            jax.ShapeDtypeStruct((B, L, D), jnp.float32),
            jax.ShapeDtypeStruct((B, L, D), jnp.float32),
        ),
        mesh=mesh,
        scratch_types=[
            pltpu.VMEM((bpw, L), jnp.int32),
            pltpu.VMEM((NBUF, CB, L, D), jnp.float32),
            pltpu.VMEM((NBUF, CB, L, D), jnp.float32),
            pltpu.SemaphoreType.DMA((NBUF,)),
            pltpu.SemaphoreType.DMA((NBUF,)),
            pltpu.SemaphoreType.DMA((NBUF,)),
            pltpu.SemaphoreType.DMA((NBUF,)),
        ],
        compiler_params=pltpu.CompilerParams(use_tc_tiling_on_sc=False),
    )
    def k(idx_hbm, amp_hbm, ph_hbm, amp_out, ph_out,
          idx_v, abuf, pbuf, ga_sem, gp_sem, oa_sem, op_sem):
        wid = lax.axis_index("s") * NC + lax.axis_index("c")
        b0 = wid * bpw

        pltpu.sync_copy(idx_hbm.at[pl.ds(b0, bpw), :], idx_v)

        def gather_descs(j, s):
            res = []
            for p in range(CB):
                ib = idx_v.at[CB * j + p]
                res.append(pltpu.make_async_copy(
                    amp_hbm.at[ib], abuf.at[s, p], ga_sem.at[s]))
                res.append(pltpu.make_async_copy(
                    ph_hbm.at[ib], pbuf.at[s, p], gp_sem.at[s]))
            return res

        def out_descs(j, s):
            sl = pl.ds(b0 + CB * j, CB)
            return (
                pltpu.make_async_copy(abuf.at[s], amp_out.at[sl], oa_sem.at[s]),
                pltpu.make_async_copy(pbuf.at[s], ph_out.at[sl], op_sem.at[s]),
            )

        for j in range(LA):
            for d in gather_descs(j, j % NBUF):
                d.start()

        def group(g, carry):
            for s in range(NBUF):
                j = g * NBUF + s
                for d in gather_descs(j, s):
                    d.wait()
                oa, op = out_descs(j, s)
                oa.start()
                op.start()
                sn = (s + LA) % NBUF
                jn = j + LA

                @pl.when(jn >= NBUF)
                def _drain():
                    poa, pop = out_descs(jn - NBUF, sn)
                    poa.wait()
                    pop.wait()

                @pl.when(jn < n_chunks)
                def _prefetch():
                    for d in gather_descs(jn, sn):
                        d.start()
            return carry

        lax.fori_loop(0, n_groups, group, 0)

        for j in range(n_chunks - (NBUF - LA), n_chunks):
            oa, op = out_descs(j, j % NBUF)
            oa.wait()
            op.wait()

    return k


def kernel(indices, amplitude_table, phase_table):
    B, L = indices.shape
    _, D = amplitude_table.shape
    return _dual_gather(B, L, D)(indices, amplitude_table, phase_table)
